# Initial kernel scaffold; baseline (speedup 1.0000x reference)
import jax
import jax.numpy as jnp
from jax.experimental import pallas as pl


def kernel(input_ids, id_map, table):
    raise NotImplementedError("write your pallas kernel here")


# speedup vs baseline: 15.1524x; 15.1524x over previous
import jax
import jax.numpy as jnp
from jax import lax
from jax.experimental import pallas as pl
from jax.experimental.pallas import tpu as pltpu
from jax.experimental.pallas import tpu_sc as plsc

VOCAB = 100000
EMBED = 64
NC, NS = 2, 16
NW = NC * NS
G = 128
K = 4
VPAD = 102400


def _remap_body(idmap_hbm, table_hbm, remap_hbm, idx_v, rows_v, sem):
    wid = lax.axis_index("s") * NC + lax.axis_index("c")
    gpw = VPAD // (G * NW)

    @pl.loop(0, gpw)
    def body(i):
        g = wid * gpw + i
        pltpu.sync_copy(idmap_hbm.at[pl.ds(g * G, G)], idx_v)
        pltpu.async_copy(table_hbm.at[idx_v], rows_v, sem).wait()
        pltpu.sync_copy(rows_v, remap_hbm.at[pl.ds(g * G, G)])


def _gather_body(ids_hbm, remap_hbm, out_hbm, idx_v, rows_v, sem):
    wid = lax.axis_index("s") * NC + lax.axis_index("c")
    ngroups = ids_hbm.shape[0]
    gpw = ngroups // NW
    nchunks = gpw // K

    @pl.loop(0, nchunks)
    def body(i):
        g0 = wid * gpw + i * K
        pltpu.sync_copy(ids_hbm.at[pl.ds(g0, K)], idx_v)
        handles = [
            pltpu.async_copy(
                remap_hbm.at[idx_v.at[j]], rows_v.at[pl.ds(j * G, G)], sem
            )
            for j in range(K)
        ]
        for h in handles:
            h.wait()
        pltpu.sync_copy(rows_v, out_hbm.at[pl.ds(g0 * G, K * G)])


def kernel(input_ids, id_map, table):
    B, H = input_ids.shape
    N = B * H
    ids = input_ids.reshape(N).astype(jnp.int32)
    idm = id_map.astype(jnp.int32)
    idm = jnp.concatenate([idm, jnp.zeros((VPAD - VOCAB,), jnp.int32)])
    ids2d = ids.reshape(N // G, G)
    table = table.astype(jnp.float32)

    mesh = plsc.VectorSubcoreMesh(core_axis_name="c", subcore_axis_name="s")
    params = pltpu.CompilerParams(use_tc_tiling_on_sc=False)

    remap = pl.kernel(
        _remap_body,
        out_type=jax.ShapeDtypeStruct((VPAD, EMBED), jnp.float32),
        mesh=mesh,
        compiler_params=params,
        scratch_types=[
            pltpu.VMEM((G,), jnp.int32),
            pltpu.VMEM((G, EMBED), jnp.float32),
            pltpu.SemaphoreType.DMA,
        ],
        name="remap_table_sc",
    )(idm, table)

    out = pl.kernel(
        _gather_body,
        out_type=jax.ShapeDtypeStruct((N, EMBED), jnp.float32),
        mesh=mesh,
        compiler_params=params,
        scratch_types=[
            pltpu.VMEM((K, G), jnp.int32),
            pltpu.VMEM((K * G, EMBED), jnp.float32),
            pltpu.SemaphoreType.DMA,
        ],
        name="token_gather_sc",
    )(ids2d, remap)

    return out.reshape(B, H, EMBED)


# speedup vs baseline: 16.8009x; 1.1088x over previous
import jax
import jax.numpy as jnp
from jax import lax
from jax.experimental import pallas as pl
from jax.experimental.pallas import tpu as pltpu
from jax.experimental.pallas import tpu_sc as plsc

VOCAB = 100000
EMBED = 64
NC, NS = 2, 16
NW = NC * NS
G = 128
K = 4
VPAD = 102400


def _remap_body(idmap_hbm, table_hbm, remap_hbm, idx_v, rows_v, sem):
    wid = lax.axis_index("s") * NC + lax.axis_index("c")
    gpw = VPAD // (G * NW)

    @pl.loop(0, gpw)
    def body(i):
        g = wid * gpw + i
        pltpu.sync_copy(idmap_hbm.at[pl.ds(g * G, G)], idx_v)
        pltpu.async_copy(table_hbm.at[idx_v], rows_v, sem).wait()
        pltpu.sync_copy(rows_v, remap_hbm.at[pl.ds(g * G, G)])


def _gather_body(ids_hbm, remap_hbm, out_hbm, idx_v, rows_v, sem):
    wid = lax.axis_index("s") * NC + lax.axis_index("c")
    ngroups = ids_hbm.shape[0]
    gpw = ngroups // NW
    nchunks = gpw // K
    CH = K * G

    pltpu.sync_copy(ids_hbm.at[pl.ds(wid * gpw, K)], idx_v.at[0])

    @pl.loop(0, nchunks, step=2)
    def outer(i0):
        for b in range(2):
            c = i0 + b
            nb = 1 - b
            handles = [
                pltpu.async_copy(
                    remap_hbm.at[idx_v.at[b, j]],
                    rows_v.at[b, pl.ds(j * G, G)],
                    sem,
                )
                for j in range(K)
            ]

            @pl.when(c + 1 < nchunks)
            def _():
                pltpu.sync_copy(
                    ids_hbm.at[pl.ds(wid * gpw + (c + 1) * K, K)], idx_v.at[nb]
                )

            @pl.when(c > 0)
            def _():
                pltpu.sync_copy(
                    rows_v.at[nb],
                    out_hbm.at[pl.ds((wid * gpw + (c - 1) * K) * G, CH)],
                )

            for h in handles:
                h.wait()

    pltpu.sync_copy(
        rows_v.at[1], out_hbm.at[pl.ds((wid * gpw + (nchunks - 1) * K) * G, CH)]
    )


def kernel(input_ids, id_map, table):
    B, H = input_ids.shape
    N = B * H
    ids = input_ids.reshape(N).astype(jnp.int32)
    idm = id_map.astype(jnp.int32)
    idm = jnp.concatenate([idm, jnp.zeros((VPAD - VOCAB,), jnp.int32)])
    ids2d = ids.reshape(N // G, G)
    table = table.astype(jnp.float32)

    mesh = plsc.VectorSubcoreMesh(core_axis_name="c", subcore_axis_name="s")
    params = pltpu.CompilerParams(use_tc_tiling_on_sc=False)

    remap = pl.kernel(
        _remap_body,
        out_type=jax.ShapeDtypeStruct((VPAD, EMBED), jnp.float32),
        mesh=mesh,
        compiler_params=params,
        scratch_types=[
            pltpu.VMEM((G,), jnp.int32),
            pltpu.VMEM((G, EMBED), jnp.float32),
            pltpu.SemaphoreType.DMA,
        ],
        name="remap_table_sc",
    )(idm, table)

    out = pl.kernel(
        _gather_body,
        out_type=jax.ShapeDtypeStruct((N, EMBED), jnp.float32),
        mesh=mesh,
        compiler_params=params,
        scratch_types=[
            pltpu.VMEM((2, K, G), jnp.int32),
            pltpu.VMEM((2, K * G, EMBED), jnp.float32),
            pltpu.SemaphoreType.DMA,
        ],
        name="token_gather_sc",
    )(ids2d, remap)

    return out.reshape(B, H, EMBED)
